# baseline (speedup 1.0000x reference)
import jax
import jax.numpy as jnp
from jax import lax
from jax.experimental import pallas as pl
from jax.experimental.pallas import tpu as pltpu
from jax.experimental.pallas import tpu_sc as plsc

_N = 20000
_NT = 16
_C = 1280
_NPAD = _NT * _C
_V = _C // 16
_BITS = 10
_RADIX = 1 << _BITS
_KEY_PASSES = -(-30 // _BITS)
_PAD_KEY = 0x3FFFFFFF
_LASTBASE = (_NT - 1) * _C
_LASTCNT = _N - _LASTBASE
_GRID = _RADIX * _NT
_SEG = _GRID // _NT
_SEGSH = _BITS + 4 - 4


def _sl(j):
    return pl.ds(pl.multiple_of(j * 16, 16), 16)


def _sort_body(c0_hbm, c1_hbm, c2_hbm, c3_hbm, lab_hbm,
               obb, olab,
               left_sp, top_sp, c2_sp, c3_sp, lab_sp, idx_a, idx_b, grid_sp,
               tot_sp,
               col_v, key_v, idx_v, rank_v, g2_v, g3_v, hist_v, offs_v,
               oidx_v, oraw_v, seg_v, tstage_v, tot_v, segb_v,
               d_v, cnt_v, last_v, flat_v, sem):
    i32 = jnp.int32
    t = lax.axis_index("s")
    base = pl.multiple_of(t * _C, _C)
    lane = lax.iota(i32, 16)
    zeros = jnp.zeros((16,), i32)
    is_last = t == _NT - 1

    def chunked(fn_full, fn_last):
        @pl.when(jnp.logical_not(is_last))
        def _():
            fn_full()

        @pl.when(is_last)
        def _():
            fn_last()

    stage = ((c0_hbm, key_v, left_sp), (c1_hbm, rank_v, top_sp),
             (c2_hbm, g2_v, c2_sp), (c3_hbm, g3_v, c3_sp),
             (lab_hbm, col_v, lab_sp))

    def loads_full():
        waits = [pltpu.async_copy(src.at[pl.ds(base, _C)], buf, sem)
                 for src, buf, _ in stage]
        for w in waits:
            w.wait()

    def loads_last():
        waits = [pltpu.async_copy(src.at[pl.ds(_LASTBASE, _LASTCNT)],
                                  buf.at[pl.ds(0, _LASTCNT)], sem)
                 for src, buf, _ in stage]
        for w in waits:
            w.wait()
    chunked(loads_full, loads_last)

    def pad_body(j, c):
        gpos = base + j * 16 + lane
        valid = gpos < _N
        key_v[_sl(j)] = jnp.where(valid, key_v[_sl(j)], _PAD_KEY)
        rank_v[_sl(j)] = jnp.where(valid, rank_v[_sl(j)], _PAD_KEY)
        idx_v[_sl(j)] = gpos
        return c
    lax.fori_loop(0, _V, pad_body, 0)

    waits = [pltpu.async_copy(buf, dst.at[pl.ds(base, _C)], sem)
             for _, buf, dst in stage]
    for w in waits:
        w.wait()

    def oidx_body(m, c):
        oidx_v[_sl(m)] = (m * 16 + lane) * _NT + t
        return c
    lax.fori_loop(0, _RADIX // 16, oidx_body, 0)

    def run_pass(keysrc_sp, shift, src_sp, dst_sp, first):
        if not first:
            pltpu.sync_copy(src_sp.at[pl.ds(base, _C)], idx_v)
            pltpu.sync_copy(keysrc_sp.at[idx_v], key_v)

        def zero_body(m, c):
            hist_v[_sl(m)] = zeros
            return c
        lax.fori_loop(0, _RADIX // 16, zero_body, 0)

        def hist_body(jj, c):
            for u in range(2):
                j = jj * 2 + u
                k = key_v[_sl(j)]
                d = lax.shift_right_logical(k, shift) & (_RADIX - 1)
                cnt, lastm = plsc.scan_count(d)
                d_v[_sl(j)] = d
                cnt_v[_sl(j)] = cnt
                last_v[_sl(j)] = jnp.where(lastm, 1, 0)
                plsc.addupdate_scatter(hist_v, [d], cnt, mask=lastm)
            return c
        lax.fori_loop(0, _V // 2, hist_body, 0)

        pltpu.sync_copy(hist_v, grid_sp.at[oidx_v])
        plsc.subcore_barrier()

        pltpu.sync_copy(grid_sp.at[pl.ds(pl.multiple_of(t * _SEG, _SEG), _SEG)],
                        seg_v)

        def scan_body(n, carry):
            g = seg_v[_sl(n)]
            s = plsc.cumsum(g)
            seg_v[_sl(n)] = s - g + carry
            return carry + jnp.max(s)
        total = lax.fori_loop(0, _SEG // 16, scan_body, jnp.int32(0))
        pltpu.sync_copy(seg_v,
                        grid_sp.at[pl.ds(pl.multiple_of(t * _SEG, _SEG), _SEG)])
        tstage_v[pl.ds(0, 16)] = jnp.broadcast_to(total, (16,))
        pltpu.sync_copy(tstage_v,
                        tot_sp.at[pl.ds(pl.multiple_of(t * 16, 16), 16)])
        plsc.subcore_barrier()

        pltpu.sync_copy(tot_sp, tot_v)
        th = plsc.load_gather(tot_v, [lane * 16])
        segb_v[pl.ds(0, 16)] = plsc.cumsum(th) - th
        pltpu.sync_copy(grid_sp.at[oidx_v], oraw_v)

        def offs_body(m, c):
            p = (m * 16 + lane) * _NT + t
            sb = plsc.load_gather(segb_v, [lax.shift_right_logical(p, _SEGSH)])
            offs_v[_sl(m)] = oraw_v[_sl(m)] + sb
            return c
        lax.fori_loop(0, _RADIX // 16, offs_body, 0)

        def perm_body(jj, c):
            for u in range(2):
                j = jj * 2 + u
                d = d_v[_sl(j)]
                cnt = cnt_v[_sl(j)]
                lastm = last_v[_sl(j)] != 0
                st = plsc.load_gather(offs_v, [d])
                rank_v[_sl(j)] = st + cnt - 1
                plsc.addupdate_scatter(offs_v, [d], cnt, mask=lastm)
            return c
        lax.fori_loop(0, _V // 2, perm_body, 0)
        pltpu.sync_copy(idx_v, dst_sp.at[rank_v])
        plsc.subcore_barrier()

    pno = 0
    nfull = 2 * _KEY_PASSES
    for keysrc in (left_sp, top_sp):
        for p in range(_KEY_PASSES):
            if pno == 0:
                src, dst = None, idx_a
            elif pno % 2 == 1:
                src, dst = idx_a, idx_b
            else:
                src, dst = idx_b, idx_a
            run_pass(keysrc, p * _BITS, src, dst, pno == 0)
            pno += 1
    final_idx = idx_b if (nfull - 1) % 2 == 1 else idx_a

    gathers = ((left_sp, key_v), (top_sp, rank_v), (c2_sp, g2_v),
               (c3_sp, g3_v), (lab_sp, col_v))
    cols = (key_v, rank_v, g2_v, g3_v)

    def interleave(nv):
        def ib(j, c):
            pos = (j * 16 + lane) * 4
            for ci, buf in enumerate(cols):
                plsc.store_scatter(flat_v, [pos + ci], buf[_sl(j)])
            return c
        lax.fori_loop(0, nv, ib, 0)

    def out_full():
        pltpu.sync_copy(final_idx.at[pl.ds(base, _C)], idx_v)
        waits = [pltpu.async_copy(sp.at[idx_v], buf, sem)
                 for sp, buf in gathers]
        for w in waits:
            w.wait()
        interleave(_V)
        w1 = pltpu.async_copy(
            flat_v, obb.at[pl.ds(pl.multiple_of(base * 4, 8), 4 * _C)], sem)
        w2 = pltpu.async_copy(col_v, olab.at[pl.ds(base, _C)], sem)
        w1.wait()
        w2.wait()

    def out_last():
        pltpu.sync_copy(final_idx.at[pl.ds(_LASTBASE, _LASTCNT)],
                        idx_v.at[pl.ds(0, _LASTCNT)])
        waits = [pltpu.async_copy(sp.at[idx_v.at[pl.ds(0, _LASTCNT)]],
                                  buf.at[pl.ds(0, _LASTCNT)], sem)
                 for sp, buf in gathers]
        for w in waits:
            w.wait()
        interleave(_LASTCNT // 16)
        w1 = pltpu.async_copy(
            flat_v.at[pl.ds(0, 4 * _LASTCNT)],
            obb.at[pl.ds(4 * _LASTBASE, 4 * _LASTCNT)], sem)
        w2 = pltpu.async_copy(col_v.at[pl.ds(0, _LASTCNT)],
                              olab.at[pl.ds(_LASTBASE, _LASTCNT)], sem)
        w1.wait()
        w2.wait()
    chunked(out_full, out_last)


_mesh = plsc.VectorSubcoreMesh(
    core_axis_name="c", subcore_axis_name="s", num_cores=1)

_sort = pl.kernel(
    _sort_body,
    out_type=(jax.ShapeDtypeStruct((_N * 4,), jnp.int32),
              jax.ShapeDtypeStruct((_N,), jnp.int32)),
    mesh=_mesh,
    compiler_params=pltpu.CompilerParams(
        needs_layout_passes=False, use_tc_tiling_on_sc=False),
    scratch_types=[
        pltpu.VMEM_SHARED((_NPAD,), jnp.int32),
        pltpu.VMEM_SHARED((_NPAD,), jnp.int32),
        pltpu.VMEM_SHARED((_NPAD,), jnp.int32),
        pltpu.VMEM_SHARED((_NPAD,), jnp.int32),
        pltpu.VMEM_SHARED((_NPAD,), jnp.int32),
        pltpu.VMEM_SHARED((_NPAD,), jnp.int32),
        pltpu.VMEM_SHARED((_NPAD,), jnp.int32),
        pltpu.VMEM_SHARED((_GRID,), jnp.int32),
        pltpu.VMEM_SHARED((_NT * 16,), jnp.int32),
        pltpu.VMEM((_C,), jnp.int32),
        pltpu.VMEM((_C,), jnp.int32),
        pltpu.VMEM((_C,), jnp.int32),
        pltpu.VMEM((_C,), jnp.int32),
        pltpu.VMEM((_C,), jnp.int32),
        pltpu.VMEM((_C,), jnp.int32),
        pltpu.VMEM((_RADIX,), jnp.int32),
        pltpu.VMEM((_RADIX,), jnp.int32),
        pltpu.VMEM((_RADIX,), jnp.int32),
        pltpu.VMEM((_RADIX,), jnp.int32),
        pltpu.VMEM((_SEG,), jnp.int32),
        pltpu.VMEM((16,), jnp.int32),
        pltpu.VMEM((_NT * 16,), jnp.int32),
        pltpu.VMEM((16,), jnp.int32),
        pltpu.VMEM((_C,), jnp.int32),
        pltpu.VMEM((_C,), jnp.int32),
        pltpu.VMEM((_C,), jnp.int32),
        pltpu.VMEM((4 * _C,), jnp.int32),
        pltpu.SemaphoreType.DMA,
    ],
)


def kernel(bboxes, labels):
    cols = [lax.bitcast_convert_type(bboxes[:, i], jnp.int32)
            for i in range(4)]
    flat, slab = _sort(cols[0], cols[1], cols[2], cols[3], labels)
    sorted_bb = lax.bitcast_convert_type(
        flat.reshape(_N, 4), jnp.float32)
    return sorted_bb, slab, sorted_bb


# speedup vs baseline: 1.3370x; 1.3370x over previous
import jax
import jax.numpy as jnp
from jax import lax
from jax.experimental import pallas as pl
from jax.experimental.pallas import tpu as pltpu
from jax.experimental.pallas import tpu_sc as plsc

_N = 20000
_NT = 16
_C = 1280
_NPAD = _NT * _C
_V = _C // 16
_BITS = 10
_RADIX = 1 << _BITS
_KEY_PASSES = -(-30 // _BITS)
_PAD_KEY = 0x3FFFFFFF
_LASTBASE = (_NT - 1) * _C
_LASTCNT = _N - _LASTBASE
_GRID = _RADIX * _NT
_SEG = _GRID // _NT
_SEGSH = _BITS + 4 - 4


def _sl(j):
    return pl.ds(pl.multiple_of(j * 16, 16), 16)


def _sort_body(c0_hbm, c1_hbm, c2_hbm, c3_hbm, lab_hbm,
               o0, o1, o2, o3, olab,
               left_sp, top_sp, c2_sp, c3_sp, lab_sp, idx_a, idx_b, grid_sp,
               tot_sp,
               col_v, key_v, idx_v, rank_v, g2_v, g3_v, hist_v, offs_v,
               oidx_v, oraw_v, seg_v, tstage_v, tot_v, segb_v,
               d_v, cnt_v, last_v, sem, sem2):
    i32 = jnp.int32
    t = lax.axis_index("s")
    base = pl.multiple_of(t * _C, _C)
    lane = lax.iota(i32, 16)
    zeros = jnp.zeros((16,), i32)
    is_last = t == _NT - 1

    def chunked(fn_full, fn_last):
        @pl.when(jnp.logical_not(is_last))
        def _():
            fn_full()

        @pl.when(is_last)
        def _():
            fn_last()

    stage = ((c0_hbm, key_v, left_sp), (c1_hbm, rank_v, top_sp),
             (c2_hbm, g2_v, c2_sp), (c3_hbm, g3_v, c3_sp),
             (lab_hbm, col_v, lab_sp))

    def loads_full():
        waits = [pltpu.async_copy(src.at[pl.ds(base, _C)], buf, sem)
                 for src, buf, _ in stage]
        for w in waits:
            w.wait()

    def loads_last():
        waits = [pltpu.async_copy(src.at[pl.ds(_LASTBASE, _LASTCNT)],
                                  buf.at[pl.ds(0, _LASTCNT)], sem)
                 for src, buf, _ in stage]
        for w in waits:
            w.wait()
    chunked(loads_full, loads_last)

    def pad_body(j, c):
        gpos = base + j * 16 + lane
        valid = gpos < _N
        key_v[_sl(j)] = jnp.where(valid, key_v[_sl(j)], _PAD_KEY)
        rank_v[_sl(j)] = jnp.where(valid, rank_v[_sl(j)], _PAD_KEY)
        idx_v[_sl(j)] = gpos
        return c
    lax.fori_loop(0, _V, pad_body, 0)

    waits = [pltpu.async_copy(buf, dst.at[pl.ds(base, _C)], sem)
             for _, buf, dst in stage]
    for w in waits:
        w.wait()

    def oidx_body(m, c):
        oidx_v[_sl(m)] = (m * 16 + lane) * _NT + t
        return c
    lax.fori_loop(0, _RADIX // 16, oidx_body, 0)

    _H = _C // 2
    _HSL = (pl.ds(0, _H), pl.ds(pl.multiple_of(_H, 8), _H))

    def run_pass(keysrc_sp, shift, src_sp, dst_sp, first):
        def hist_range(lo, hi):
            def hist_body(jj, c):
                for u in range(4):
                    j = jj * 4 + u
                    k = key_v[_sl(j)]
                    d = lax.shift_right_logical(k, shift) & (_RADIX - 1)
                    cnt, lastm = plsc.scan_count(d)
                    d_v[_sl(j)] = d
                    cnt_v[_sl(j)] = cnt
                    last_v[_sl(j)] = jnp.where(lastm, 1, 0)
                    plsc.addupdate_scatter(hist_v, [d], cnt, mask=lastm)
                return c
            lax.fori_loop(lo // 64, hi // 64, hist_body, 0)

        def zero_body(m, c):
            hist_v[_sl(m)] = zeros
            return c

        if not first:
            w1 = pltpu.async_copy(src_sp.at[pl.ds(base, _H)],
                                  idx_v.at[_HSL[0]], sem)
            w2 = pltpu.async_copy(
                src_sp.at[pl.ds(pl.multiple_of(base + _H, 8), _H)],
                idx_v.at[_HSL[1]], sem2)
            w1.wait()
            g1 = pltpu.async_copy(keysrc_sp.at[idx_v.at[_HSL[0]]],
                                  key_v.at[_HSL[0]], sem)
            w2.wait()
            g2 = pltpu.async_copy(keysrc_sp.at[idx_v.at[_HSL[1]]],
                                  key_v.at[_HSL[1]], sem2)
            lax.fori_loop(0, _RADIX // 16, zero_body, 0)
            g1.wait()
            hist_range(0, _H)
            g2.wait()
            hist_range(_H, _C)
        else:
            lax.fori_loop(0, _RADIX // 16, zero_body, 0)
            hist_range(0, _C)

        pltpu.sync_copy(hist_v, grid_sp.at[oidx_v])
        plsc.subcore_barrier()

        pltpu.sync_copy(grid_sp.at[pl.ds(pl.multiple_of(t * _SEG, _SEG), _SEG)],
                        seg_v)

        def scan_body(n, carry):
            g = seg_v[_sl(n)]
            s = plsc.cumsum(g)
            seg_v[_sl(n)] = s - g + carry
            return carry + jnp.max(s)
        total = lax.fori_loop(0, _SEG // 16, scan_body, jnp.int32(0))
        pltpu.sync_copy(seg_v,
                        grid_sp.at[pl.ds(pl.multiple_of(t * _SEG, _SEG), _SEG)])
        tstage_v[pl.ds(0, 16)] = jnp.broadcast_to(total, (16,))
        pltpu.sync_copy(tstage_v,
                        tot_sp.at[pl.ds(pl.multiple_of(t * 16, 16), 16)])
        plsc.subcore_barrier()

        pltpu.sync_copy(tot_sp, tot_v)
        th = plsc.load_gather(tot_v, [lane * 16])
        segb_v[pl.ds(0, 16)] = plsc.cumsum(th) - th
        pltpu.sync_copy(grid_sp.at[oidx_v], oraw_v)

        def offs_body(m, c):
            p = (m * 16 + lane) * _NT + t
            sb = plsc.load_gather(segb_v, [lax.shift_right_logical(p, _SEGSH)])
            offs_v[_sl(m)] = oraw_v[_sl(m)] + sb
            return c
        lax.fori_loop(0, _RADIX // 16, offs_body, 0)

        def perm_body(jj, c):
            for u in range(4):
                j = jj * 4 + u
                d = d_v[_sl(j)]
                cnt = cnt_v[_sl(j)]
                lastm = last_v[_sl(j)] != 0
                st = plsc.load_gather(offs_v, [d])
                rank_v[_sl(j)] = st + cnt - 1
                plsc.addupdate_scatter(offs_v, [d], cnt, mask=lastm)
            return c
        lax.fori_loop(0, _V // 4, perm_body, 0)
        pltpu.sync_copy(idx_v, dst_sp.at[rank_v])
        plsc.subcore_barrier()

    pno = 0
    nfull = 2 * _KEY_PASSES
    for keysrc in (left_sp, top_sp):
        for p in range(_KEY_PASSES):
            if pno == 0:
                src, dst = None, idx_a
            elif pno % 2 == 1:
                src, dst = idx_a, idx_b
            else:
                src, dst = idx_b, idx_a
            run_pass(keysrc, p * _BITS, src, dst, pno == 0)
            pno += 1
    final_idx = idx_b if (nfull - 1) % 2 == 1 else idx_a

    gathers = ((left_sp, key_v, o0), (top_sp, rank_v, o1), (c2_sp, g2_v, o2),
               (c3_sp, g3_v, o3), (lab_sp, col_v, olab))

    def out_full():
        pltpu.sync_copy(final_idx.at[pl.ds(base, _C)], idx_v)
        waits = [pltpu.async_copy(sp.at[idx_v], buf, sem)
                 for sp, buf, _ in gathers]
        for w in waits:
            w.wait()
        waits = [pltpu.async_copy(buf, out.at[pl.ds(base, _C)], sem)
                 for _, buf, out in gathers]
        for w in waits:
            w.wait()

    def out_last():
        pltpu.sync_copy(final_idx.at[pl.ds(_LASTBASE, _LASTCNT)],
                        idx_v.at[pl.ds(0, _LASTCNT)])
        waits = [pltpu.async_copy(sp.at[idx_v.at[pl.ds(0, _LASTCNT)]],
                                  buf.at[pl.ds(0, _LASTCNT)], sem)
                 for sp, buf, _ in gathers]
        for w in waits:
            w.wait()
        waits = [pltpu.async_copy(buf.at[pl.ds(0, _LASTCNT)],
                                  out.at[pl.ds(_LASTBASE, _LASTCNT)], sem)
                 for _, buf, out in gathers]
        for w in waits:
            w.wait()
    chunked(out_full, out_last)


_mesh = plsc.VectorSubcoreMesh(
    core_axis_name="c", subcore_axis_name="s", num_cores=1)

_i32col = jax.ShapeDtypeStruct((_N,), jnp.int32)

_sort = pl.kernel(
    _sort_body,
    out_type=(_i32col,) * 5,
    mesh=_mesh,
    compiler_params=pltpu.CompilerParams(
        needs_layout_passes=False, use_tc_tiling_on_sc=False),
    scratch_types=[
        pltpu.VMEM_SHARED((_NPAD,), jnp.int32),
        pltpu.VMEM_SHARED((_NPAD,), jnp.int32),
        pltpu.VMEM_SHARED((_NPAD,), jnp.int32),
        pltpu.VMEM_SHARED((_NPAD,), jnp.int32),
        pltpu.VMEM_SHARED((_NPAD,), jnp.int32),
        pltpu.VMEM_SHARED((_NPAD,), jnp.int32),
        pltpu.VMEM_SHARED((_NPAD,), jnp.int32),
        pltpu.VMEM_SHARED((_GRID,), jnp.int32),
        pltpu.VMEM_SHARED((_NT * 16,), jnp.int32),
        pltpu.VMEM((_C,), jnp.int32),
        pltpu.VMEM((_C,), jnp.int32),
        pltpu.VMEM((_C,), jnp.int32),
        pltpu.VMEM((_C,), jnp.int32),
        pltpu.VMEM((_C,), jnp.int32),
        pltpu.VMEM((_C,), jnp.int32),
        pltpu.VMEM((_RADIX,), jnp.int32),
        pltpu.VMEM((_RADIX,), jnp.int32),
        pltpu.VMEM((_RADIX,), jnp.int32),
        pltpu.VMEM((_RADIX,), jnp.int32),
        pltpu.VMEM((_SEG,), jnp.int32),
        pltpu.VMEM((16,), jnp.int32),
        pltpu.VMEM((_NT * 16,), jnp.int32),
        pltpu.VMEM((16,), jnp.int32),
        pltpu.VMEM((_C,), jnp.int32),
        pltpu.VMEM((_C,), jnp.int32),
        pltpu.VMEM((_C,), jnp.int32),
        pltpu.SemaphoreType.DMA,
        pltpu.SemaphoreType.DMA,
    ],
)


def kernel(bboxes, labels):
    cols = [lax.bitcast_convert_type(bboxes[:, i], jnp.int32)
            for i in range(4)]
    s0, s1, s2, s3, slab = _sort(cols[0], cols[1], cols[2], cols[3], labels)
    sorted_bb = lax.bitcast_convert_type(
        jnp.stack([s0, s1, s2, s3], axis=1), jnp.float32)
    return sorted_bb, slab, sorted_bb


# speedup vs baseline: 1.3419x; 1.0036x over previous
import jax
import jax.numpy as jnp
from jax import lax
from jax.experimental import pallas as pl
from jax.experimental.pallas import tpu as pltpu
from jax.experimental.pallas import tpu_sc as plsc

_N = 20000
_NT = 16
_C = 1280
_NPAD = _NT * _C
_V = _C // 16
_BITS = 10
_RADIX = 1 << _BITS
_KEY_PASSES = -(-30 // _BITS)
_PAD_KEY = 0x3FFFFFFF
_LASTBASE = (_NT - 1) * _C
_LASTCNT = _N - _LASTBASE
_GRID = _RADIX * _NT
_SEG = _GRID // _NT
_SEGSH = _BITS + 4 - 4


def _sl(j):
    return pl.ds(pl.multiple_of(j * 16, 16), 16)


def _sort_body(c0_hbm, c1_hbm, c2_hbm, c3_hbm, lab_hbm,
               o0, o1, o2, o3, olab,
               left_sp, top_sp, c2_sp, c3_sp, lab_sp, idx_a, idx_b, grid_sp,
               tot_sp,
               col_v, key_v, idx_v, rank_v, g2_v, g3_v, hist_v, offs_v,
               oidx_v, oraw_v, seg_v, tstage_v, tot_v, segb_v,
               d_v, cnt_v, last_v, sem, sem2):
    i32 = jnp.int32
    t = lax.axis_index("s")
    base = pl.multiple_of(t * _C, _C)
    lane = lax.iota(i32, 16)
    zeros = jnp.zeros((16,), i32)
    is_last = t == _NT - 1

    def chunked(fn_full, fn_last):
        @pl.when(jnp.logical_not(is_last))
        def _():
            fn_full()

        @pl.when(is_last)
        def _():
            fn_last()

    stage = ((c0_hbm, key_v, left_sp), (c1_hbm, rank_v, top_sp),
             (c2_hbm, g2_v, c2_sp), (c3_hbm, g3_v, c3_sp),
             (lab_hbm, col_v, lab_sp))

    def loads_full():
        waits = [pltpu.async_copy(src.at[pl.ds(base, _C)], buf, sem)
                 for src, buf, _ in stage]
        for w in waits:
            w.wait()

    def loads_last():
        waits = [pltpu.async_copy(src.at[pl.ds(_LASTBASE, _LASTCNT)],
                                  buf.at[pl.ds(0, _LASTCNT)], sem)
                 for src, buf, _ in stage]
        for w in waits:
            w.wait()
    chunked(loads_full, loads_last)

    def pad_body(j, c):
        gpos = base + j * 16 + lane
        valid = gpos < _N
        key_v[_sl(j)] = jnp.where(valid, key_v[_sl(j)], _PAD_KEY)
        rank_v[_sl(j)] = jnp.where(valid, rank_v[_sl(j)], _PAD_KEY)
        idx_v[_sl(j)] = gpos
        return c
    lax.fori_loop(0, _V, pad_body, 0)

    stage_waits = [pltpu.async_copy(buf, dst.at[pl.ds(base, _C)], sem)
                   for _, buf, dst in stage]

    def oidx_body(m, c):
        oidx_v[_sl(m)] = (m * 16 + lane) * _NT + t
        return c
    lax.fori_loop(0, _RADIX // 16, oidx_body, 0)

    _H = _C // 2
    _HSL = (pl.ds(0, _H), pl.ds(pl.multiple_of(_H, 8), _H))

    def run_pass(keysrc_sp, shift, src_sp, dst_sp, first, drain=()):
        def hist_range(lo, hi):
            def hist_body(jj, c):
                for u in range(4):
                    j = jj * 4 + u
                    k = key_v[_sl(j)]
                    d = lax.shift_right_logical(k, shift) & (_RADIX - 1)
                    cnt, lastm = plsc.scan_count(d)
                    d_v[_sl(j)] = d
                    cnt_v[_sl(j)] = cnt
                    last_v[_sl(j)] = jnp.where(lastm, 1, 0)
                    plsc.addupdate_scatter(hist_v, [d], cnt, mask=lastm)
                return c
            lax.fori_loop(lo // 64, hi // 64, hist_body, 0)

        def zero_body(m, c):
            hist_v[_sl(m)] = zeros
            return c

        if not first:
            w1 = pltpu.async_copy(src_sp.at[pl.ds(base, _H)],
                                  idx_v.at[_HSL[0]], sem)
            w2 = pltpu.async_copy(
                src_sp.at[pl.ds(pl.multiple_of(base + _H, 8), _H)],
                idx_v.at[_HSL[1]], sem2)
            w1.wait()
            g1 = pltpu.async_copy(keysrc_sp.at[idx_v.at[_HSL[0]]],
                                  key_v.at[_HSL[0]], sem)
            w2.wait()
            g2 = pltpu.async_copy(keysrc_sp.at[idx_v.at[_HSL[1]]],
                                  key_v.at[_HSL[1]], sem2)
            lax.fori_loop(0, _RADIX // 16, zero_body, 0)
            g1.wait()
            hist_range(0, _H)
            g2.wait()
            hist_range(_H, _C)
        else:
            lax.fori_loop(0, _RADIX // 16, zero_body, 0)
            hist_range(0, _C)

        pltpu.sync_copy(hist_v, grid_sp.at[oidx_v])
        plsc.subcore_barrier()

        pltpu.sync_copy(grid_sp.at[pl.ds(pl.multiple_of(t * _SEG, _SEG), _SEG)],
                        seg_v)

        def scan_body(n, carry):
            g = seg_v[_sl(n)]
            s = plsc.cumsum(g)
            seg_v[_sl(n)] = s - g + carry
            return carry + jnp.max(s)
        total = lax.fori_loop(0, _SEG // 16, scan_body, jnp.int32(0))
        pltpu.sync_copy(seg_v,
                        grid_sp.at[pl.ds(pl.multiple_of(t * _SEG, _SEG), _SEG)])
        tstage_v[pl.ds(0, 16)] = jnp.broadcast_to(total, (16,))
        pltpu.sync_copy(tstage_v,
                        tot_sp.at[pl.ds(pl.multiple_of(t * 16, 16), 16)])
        plsc.subcore_barrier()

        pltpu.sync_copy(tot_sp, tot_v)
        th = plsc.load_gather(tot_v, [lane * 16])
        segb_v[pl.ds(0, 16)] = plsc.cumsum(th) - th
        pltpu.sync_copy(grid_sp.at[oidx_v], oraw_v)

        def offs_body(m, c):
            p = (m * 16 + lane) * _NT + t
            sb = plsc.load_gather(segb_v, [lax.shift_right_logical(p, _SEGSH)])
            offs_v[_sl(m)] = oraw_v[_sl(m)] + sb
            return c
        lax.fori_loop(0, _RADIX // 16, offs_body, 0)

        def perm_body(jj, c):
            for u in range(4):
                j = jj * 4 + u
                d = d_v[_sl(j)]
                cnt = cnt_v[_sl(j)]
                lastm = last_v[_sl(j)] != 0
                st = plsc.load_gather(offs_v, [d])
                rank_v[_sl(j)] = st + cnt - 1
                plsc.addupdate_scatter(offs_v, [d], cnt, mask=lastm)
            return c
        for w in drain:
            w.wait()
        lax.fori_loop(0, _V // 4, perm_body, 0)
        pltpu.sync_copy(idx_v, dst_sp.at[rank_v])
        plsc.subcore_barrier()

    pno = 0
    nfull = 2 * _KEY_PASSES
    for keysrc in (left_sp, top_sp):
        for p in range(_KEY_PASSES):
            if pno == 0:
                src, dst = None, idx_a
            elif pno % 2 == 1:
                src, dst = idx_a, idx_b
            else:
                src, dst = idx_b, idx_a
            run_pass(keysrc, p * _BITS, src, dst, pno == 0,
                     drain=(stage_waits if pno == 0 else ()))
            pno += 1
    final_idx = idx_b if (nfull - 1) % 2 == 1 else idx_a

    gathers = ((left_sp, key_v, o0), (top_sp, rank_v, o1), (c2_sp, g2_v, o2),
               (c3_sp, g3_v, o3), (lab_sp, col_v, olab))

    def out_full():
        pltpu.sync_copy(final_idx.at[pl.ds(base, _C)], idx_v)
        waits = [pltpu.async_copy(sp.at[idx_v], buf, sem)
                 for sp, buf, _ in gathers]
        for w in waits:
            w.wait()
        waits = [pltpu.async_copy(buf, out.at[pl.ds(base, _C)], sem)
                 for _, buf, out in gathers]
        for w in waits:
            w.wait()

    def out_last():
        pltpu.sync_copy(final_idx.at[pl.ds(_LASTBASE, _LASTCNT)],
                        idx_v.at[pl.ds(0, _LASTCNT)])
        waits = [pltpu.async_copy(sp.at[idx_v.at[pl.ds(0, _LASTCNT)]],
                                  buf.at[pl.ds(0, _LASTCNT)], sem)
                 for sp, buf, _ in gathers]
        for w in waits:
            w.wait()
        waits = [pltpu.async_copy(buf.at[pl.ds(0, _LASTCNT)],
                                  out.at[pl.ds(_LASTBASE, _LASTCNT)], sem)
                 for _, buf, out in gathers]
        for w in waits:
            w.wait()
    chunked(out_full, out_last)


_mesh = plsc.VectorSubcoreMesh(
    core_axis_name="c", subcore_axis_name="s", num_cores=1)

_i32col = jax.ShapeDtypeStruct((_N,), jnp.int32)

_sort = pl.kernel(
    _sort_body,
    out_type=(_i32col,) * 5,
    mesh=_mesh,
    compiler_params=pltpu.CompilerParams(
        needs_layout_passes=False, use_tc_tiling_on_sc=False),
    scratch_types=[
        pltpu.VMEM_SHARED((_NPAD,), jnp.int32),
        pltpu.VMEM_SHARED((_NPAD,), jnp.int32),
        pltpu.VMEM_SHARED((_NPAD,), jnp.int32),
        pltpu.VMEM_SHARED((_NPAD,), jnp.int32),
        pltpu.VMEM_SHARED((_NPAD,), jnp.int32),
        pltpu.VMEM_SHARED((_NPAD,), jnp.int32),
        pltpu.VMEM_SHARED((_NPAD,), jnp.int32),
        pltpu.VMEM_SHARED((_GRID,), jnp.int32),
        pltpu.VMEM_SHARED((_NT * 16,), jnp.int32),
        pltpu.VMEM((_C,), jnp.int32),
        pltpu.VMEM((_C,), jnp.int32),
        pltpu.VMEM((_C,), jnp.int32),
        pltpu.VMEM((_C,), jnp.int32),
        pltpu.VMEM((_C,), jnp.int32),
        pltpu.VMEM((_C,), jnp.int32),
        pltpu.VMEM((_RADIX,), jnp.int32),
        pltpu.VMEM((_RADIX,), jnp.int32),
        pltpu.VMEM((_RADIX,), jnp.int32),
        pltpu.VMEM((_RADIX,), jnp.int32),
        pltpu.VMEM((_SEG,), jnp.int32),
        pltpu.VMEM((16,), jnp.int32),
        pltpu.VMEM((_NT * 16,), jnp.int32),
        pltpu.VMEM((16,), jnp.int32),
        pltpu.VMEM((_C,), jnp.int32),
        pltpu.VMEM((_C,), jnp.int32),
        pltpu.VMEM((_C,), jnp.int32),
        pltpu.SemaphoreType.DMA,
        pltpu.SemaphoreType.DMA,
    ],
)


def kernel(bboxes, labels):
    cols = [lax.bitcast_convert_type(bboxes[:, i], jnp.int32)
            for i in range(4)]
    s0, s1, s2, s3, slab = _sort(cols[0], cols[1], cols[2], cols[3], labels)
    sorted_bb = lax.bitcast_convert_type(
        jnp.stack([s0, s1, s2, s3], axis=1), jnp.float32)
    return sorted_bb, slab, sorted_bb
